# baseline (speedup 1.0000x reference)
import functools

import jax
import jax.numpy as jnp
from jax import lax
from jax.experimental import pallas as pl
from jax.experimental.pallas import tpu as pltpu
from jax.experimental.pallas import tpu_sc as plsc

D_MODEL = 128
NUM_CORES = 2
NUM_SUBCORES = 16
NUM_WORKERS = NUM_CORES * NUM_SUBCORES

CHUNK = 80
NBUF = 8
LOOK = 4


@functools.partial(jax.jit, static_argnames=("b_total",))
def _embed(x_flat, table, b_total):
    b_per_w = b_total // NUM_WORKERS
    n_chunks = b_per_w // CHUNK
    n_outer = n_chunks // NBUF

    mesh = plsc.VectorSubcoreMesh(
        core_axis_name="c", subcore_axis_name="s",
        num_cores=NUM_CORES, num_subcores=NUM_SUBCORES)

    @functools.partial(
        pl.kernel,
        mesh=mesh,
        out_type=jax.ShapeDtypeStruct((b_total, D_MODEL), jnp.float32),
        scratch_types=[
            pltpu.VMEM((b_per_w,), jnp.int32),
            pltpu.VMEM((NBUF, CHUNK, D_MODEL), jnp.float32),
            pltpu.SemaphoreType.DMA,
            pltpu.SemaphoreType.DMA,
        ],
    )
    def emb(idx_hbm, table_hbm, out_hbm, idx_v, rows_v, gsem, wsem):
        wid = lax.axis_index("s") * NUM_CORES + lax.axis_index("c")
        base = wid * b_per_w
        pltpu.sync_copy(idx_hbm.at[pl.ds(base, b_per_w)], idx_v)

        def start_gather(g, slot):
            pltpu.async_copy(
                table_hbm.at[idx_v.at[pl.ds(g * CHUNK, CHUNK)]],
                rows_v.at[slot], gsem)

        def wait_gather(slot):
            pltpu.make_async_copy(
                table_hbm.at[pl.ds(0, CHUNK)], rows_v.at[slot], gsem).wait()

        def wait_write(slot):
            pltpu.make_async_copy(
                rows_v.at[slot], out_hbm.at[pl.ds(base, CHUNK)], wsem).wait()

        for b in range(LOOK):
            start_gather(b, b)

        def outer(o, carry):
            for b in range(NBUF):
                g = o * NBUF + b
                gl = g + LOOK
                sl = (b + LOOK) % NBUF

                @pl.when(jnp.logical_and(gl < n_chunks, gl >= NBUF))
                def _():
                    wait_write(sl)

                @pl.when(gl < n_chunks)
                def _():
                    start_gather(gl, sl)

                wait_gather(b)
                pltpu.async_copy(
                    rows_v.at[b],
                    out_hbm.at[pl.ds(base + g * CHUNK, CHUNK)], wsem)
            return carry

        lax.fori_loop(0, n_outer, outer, 0)

        for b in range(NBUF):
            wait_write(b)

    return emb(x_flat, table)


TC_ROWS = 20480


def kernel(x, table):
    b, s = x.shape
    n = b * s
    x_flat = x.reshape(n).astype(jnp.int32)
    sc_out = _embed(x_flat[: n - TC_ROWS], table, n - TC_ROWS)
    tc_out = jnp.take(table, x_flat[n - TC_ROWS:], axis=0)
    out = jnp.concatenate([sc_out, tc_out], axis=0)
    return out.reshape(b, s, D_MODEL)


# speedup vs baseline: 1.8434x; 1.8434x over previous
import functools

import jax
import jax.numpy as jnp
from jax import lax
from jax.experimental import pallas as pl
from jax.experimental.pallas import tpu as pltpu
from jax.experimental.pallas import tpu_sc as plsc

D_MODEL = 128
NUM_CORES = 2
NUM_SUBCORES = 16
NUM_WORKERS = NUM_CORES * NUM_SUBCORES

CHUNK = 128
NBUF = 5
LOOK = 2


@functools.partial(jax.jit, static_argnames=("b_total",))
def _embed(x_flat, table, b_total):
    b_per_w = b_total // NUM_WORKERS
    n_chunks = b_per_w // CHUNK
    n_outer = n_chunks // NBUF

    mesh = plsc.VectorSubcoreMesh(
        core_axis_name="c", subcore_axis_name="s",
        num_cores=NUM_CORES, num_subcores=NUM_SUBCORES)

    @functools.partial(
        pl.kernel,
        mesh=mesh,
        out_type=jax.ShapeDtypeStruct((b_total, D_MODEL), jnp.float32),
        scratch_types=[
            pltpu.VMEM((b_per_w,), jnp.int32),
            pltpu.VMEM((NBUF, CHUNK, D_MODEL), jnp.float32),
            pltpu.SemaphoreType.DMA,
            pltpu.SemaphoreType.DMA,
        ],
    )
    def emb(idx_hbm, table_hbm, out_hbm, idx_v, rows_v, gsem, wsem):
        wid = lax.axis_index("s") * NUM_CORES + lax.axis_index("c")
        base = wid * b_per_w
        pltpu.sync_copy(idx_hbm.at[pl.ds(base, b_per_w)], idx_v)

        def start_gather(g, slot):
            pltpu.async_copy(
                table_hbm.at[idx_v.at[pl.ds(g * CHUNK, CHUNK)]],
                rows_v.at[slot], gsem)

        def wait_gather(slot):
            pltpu.make_async_copy(
                table_hbm.at[pl.ds(0, CHUNK)], rows_v.at[slot], gsem).wait()

        def wait_write(slot):
            pltpu.make_async_copy(
                rows_v.at[slot], out_hbm.at[pl.ds(base, CHUNK)], wsem).wait()

        for b in range(LOOK):
            start_gather(b, b)

        def outer(o, carry):
            for b in range(NBUF):
                g = o * NBUF + b
                gl = g + LOOK
                sl = (b + LOOK) % NBUF

                @pl.when(jnp.logical_and(gl < n_chunks, gl >= NBUF))
                def _():
                    wait_write(sl)

                @pl.when(gl < n_chunks)
                def _():
                    start_gather(gl, sl)

                wait_gather(b)
                pltpu.async_copy(
                    rows_v.at[b],
                    out_hbm.at[pl.ds(base + g * CHUNK, CHUNK)], wsem)
            return carry

        lax.fori_loop(0, n_outer, outer, 0)

        for b in range(NBUF):
            wait_write(b)

    return emb(x_flat, table)


def kernel(x, table):
    b, s = x.shape
    x_flat = x.reshape(b * s).astype(jnp.int32)
    out = _embed(x_flat, table, b * s)
    return out.reshape(b, s, D_MODEL)
